# Initial kernel scaffold; baseline (speedup 1.0000x reference)
import functools

import jax
import jax.numpy as jnp
from jax import lax
from jax.experimental import pallas as pl
from jax.experimental.pallas import tpu as pltpu
from jax.experimental.pallas import tpu_sc as plsc

N = 10000
NP = 10240
E = 320000
NC = 2
NS = 16
NW = NC * NS
CHUNK = 128
ROWS_PER_TILE = 79
EP = NW * ROWS_PER_TILE * CHUNK
STRIPE = NP // NS
F = 16

_mesh = plsc.VectorSubcoreMesh(core_axis_name="c", subcore_axis_name="s")


@functools.partial(
    pl.kernel,
    mesh=_mesh,
    out_type=jax.ShapeDtypeStruct((NC, NP), jnp.float32),
    scratch_types=[
        pltpu.VMEM((ROWS_PER_TILE, CHUNK), jnp.int32),
        pltpu.VMEM((CHUNK,), jnp.float32),
        pltpu.VMEM((STRIPE,), jnp.float32),
        pltpu.VMEM_SHARED((NP,), jnp.float32),
        pltpu.SemaphoreType.DMA,
    ],
)
def _deg_kernel(dst_hbm, out_hbm, dst_v, ones_v, zb_v, deg_sh, sem):
    c = lax.axis_index("c")
    s = lax.axis_index("s")
    slab = c * NS + s
    for i in range(CHUNK // 16):
        ones_v[pl.ds(i * 16, 16)] = jnp.full((16,), 1.0, jnp.float32)
    for i in range(STRIPE // 16):
        zb_v[pl.ds(i * 16, 16)] = jnp.zeros((16,), jnp.float32)
    pltpu.sync_copy(zb_v, deg_sh.at[pl.ds(s * STRIPE, STRIPE)])
    plsc.subcore_barrier()
    pltpu.sync_copy(dst_hbm.at[slab], dst_v)

    def body(j, carry):
        pltpu.sync_copy(ones_v, deg_sh.at[dst_v.at[j]], add=True)
        return carry

    lax.fori_loop(0, ROWS_PER_TILE, body, 0)
    plsc.subcore_barrier()
    pltpu.sync_copy(deg_sh.at[pl.ds(s * STRIPE, STRIPE)], zb_v)
    pltpu.sync_copy(zb_v, out_hbm.at[c, pl.ds(s * STRIPE, STRIPE)])


@functools.partial(
    pl.kernel,
    mesh=_mesh,
    out_type=jax.ShapeDtypeStruct((NC, NP, F), jnp.float32),
    scratch_types=[
        pltpu.VMEM((ROWS_PER_TILE, CHUNK), jnp.int32),
        pltpu.VMEM((ROWS_PER_TILE, CHUNK), jnp.int32),
        pltpu.VMEM((CHUNK, F), jnp.float32),
        pltpu.VMEM((STRIPE, F), jnp.float32),
        pltpu.VMEM_SHARED((NP, F), jnp.float32),
        pltpu.SemaphoreType.DMA,
    ],
)
def _edge_kernel(src_hbm, dst_hbm, h_hbm, out_hbm,
                 src_v, dst_v, rows_v, zb_v, acc_sh, sem):
    c = lax.axis_index("c")
    s = lax.axis_index("s")
    slab = c * NS + s
    for i in range(CHUNK):
        rows_v[i] = jnp.zeros((F,), jnp.float32)
    for k in range(STRIPE // CHUNK):
        pltpu.sync_copy(rows_v, acc_sh.at[pl.ds(s * STRIPE + k * CHUNK, CHUNK)])
    plsc.subcore_barrier()
    pltpu.sync_copy(src_hbm.at[slab], src_v)
    pltpu.sync_copy(dst_hbm.at[slab], dst_v)

    def body(j, carry):
        pltpu.async_copy(h_hbm.at[src_v.at[j]], rows_v, sem).wait()
        pltpu.sync_copy(rows_v, acc_sh.at[dst_v.at[j]], add=True)
        return carry

    lax.fori_loop(0, ROWS_PER_TILE, body, 0)
    plsc.subcore_barrier()
    pltpu.sync_copy(acc_sh.at[pl.ds(s * STRIPE, STRIPE)], zb_v)
    pltpu.sync_copy(zb_v, out_hbm.at[c, pl.ds(s * STRIPE, STRIPE)])


def _tc1_body(x_ref, w_ref, dg_ref, dinv_ref, h_ref):
    deg = dg_ref[:, 0:1] + dg_ref[:, 1:2] + 1.0
    dinv = lax.rsqrt(deg)
    dinv_ref[...] = dinv
    h = jnp.dot(x_ref[...], w_ref[...], preferred_element_type=jnp.float32)
    h_ref[...] = h * dinv


def _tc2_body(a_ref, h_ref, dinv_ref, b_ref, w_ref, out_ref):
    dinv = dinv_ref[...]
    s1 = dinv * (a_ref[0] + a_ref[1] + h_ref[...]) + b_ref[...]
    z1 = jnp.maximum(s1, 0.0)
    out_ref[...] = jnp.dot(z1, w_ref[...],
                           preferred_element_type=jnp.float32) * dinv


def _tc3_body(a_ref, h_ref, dinv_ref, b_ref, out_ref):
    dinv = dinv_ref[...]
    s2 = dinv * (a_ref[0] + a_ref[1] + h_ref[...]) + b_ref[...]
    logits = s2[:N, 0:3]
    m = jnp.max(logits, axis=1, keepdims=True)
    e = jnp.exp(logits - m)
    se = jnp.sum(e, axis=1, keepdims=True)
    out_ref[...] = (logits - m) - jnp.log(se)


def kernel(x, edge_index, W1, b1, W2, b2):
    src = edge_index[0].astype(jnp.int32)
    dst = edge_index[1].astype(jnp.int32)
    pad = jnp.full((EP - E,), N, jnp.int32)
    src3 = jnp.concatenate([src, pad]).reshape(NW, ROWS_PER_TILE, CHUNK)
    dst3 = jnp.concatenate([dst, pad]).reshape(NW, ROWS_PER_TILE, CHUNK)
    xp = jnp.pad(x, ((0, NP - N), (0, 0)))
    w2p = jnp.pad(W2, ((0, 0), (0, F - W2.shape[1])))
    b1r = b1.reshape(1, F)
    b2r = jnp.pad(b2, (0, F - b2.shape[0])).reshape(1, F)

    deg_parts = _deg_kernel(dst3)
    deg_t = deg_parts.T

    dinv, h1p = pl.pallas_call(
        _tc1_body,
        out_shape=(jax.ShapeDtypeStruct((NP, 1), jnp.float32),
                   jax.ShapeDtypeStruct((NP, F), jnp.float32)),
    )(xp, W1, deg_t)

    acc1 = _edge_kernel(src3, dst3, h1p)

    h2p = pl.pallas_call(
        _tc2_body,
        out_shape=jax.ShapeDtypeStruct((NP, F), jnp.float32),
    )(acc1, h1p, dinv, b1r, w2p)

    acc2 = _edge_kernel(src3, dst3, h2p)

    out = pl.pallas_call(
        _tc3_body,
        out_shape=jax.ShapeDtypeStruct((N, 3), jnp.float32),
    )(acc2, h2p, dinv, b2r)
    return out


# speedup vs baseline: 35.5343x; 35.5343x over previous
import functools

import jax
import jax.numpy as jnp
from jax import lax
from jax.experimental import pallas as pl
from jax.experimental.pallas import tpu as pltpu
from jax.experimental.pallas import tpu_sc as plsc

N = 10000
NP = 10240
E = 320000
NC = 2
NS = 16
NW = NC * NS
CHUNK = 128
ROWS_PER_TILE = 79
EP = NW * ROWS_PER_TILE * CHUNK
STRIPE = NP // NS
F = 16

_mesh = plsc.VectorSubcoreMesh(core_axis_name="c", subcore_axis_name="s")
_sc_params = pltpu.CompilerParams(use_tc_tiling_on_sc=False)


@functools.partial(
    pl.kernel,
    mesh=_mesh,
    compiler_params=_sc_params,
    out_type=jax.ShapeDtypeStruct((NC, NP), jnp.float32),
    scratch_types=[
        pltpu.VMEM((ROWS_PER_TILE, CHUNK), jnp.int32),
        pltpu.VMEM((CHUNK,), jnp.float32),
        pltpu.VMEM((STRIPE,), jnp.float32),
        pltpu.VMEM_SHARED((NP,), jnp.float32),
        pltpu.SemaphoreType.DMA,
    ],
)
def _deg_kernel(dst_hbm, out_hbm, dst_v, ones_v, zb_v, deg_sh, sem):
    c = lax.axis_index("c")
    s = lax.axis_index("s")
    slab = c * NS + s
    for i in range(CHUNK // 16):
        ones_v[pl.ds(i * 16, 16)] = jnp.full((16,), 1.0, jnp.float32)
    for i in range(STRIPE // 16):
        zb_v[pl.ds(i * 16, 16)] = jnp.zeros((16,), jnp.float32)
    pltpu.sync_copy(zb_v, deg_sh.at[pl.ds(s * STRIPE, STRIPE)])
    plsc.subcore_barrier()
    pltpu.sync_copy(dst_hbm.at[slab], dst_v)

    def body(j, carry):
        pltpu.sync_copy(ones_v, deg_sh.at[dst_v.at[j]], add=True)
        return carry

    lax.fori_loop(0, ROWS_PER_TILE, body, 0)
    plsc.subcore_barrier()
    pltpu.sync_copy(deg_sh.at[pl.ds(s * STRIPE, STRIPE)], zb_v)
    pltpu.sync_copy(zb_v, out_hbm.at[c, pl.ds(s * STRIPE, STRIPE)])


@functools.partial(
    pl.kernel,
    mesh=_mesh,
    compiler_params=_sc_params,
    out_type=jax.ShapeDtypeStruct((NC, NP, F), jnp.float32),
    scratch_types=[
        pltpu.VMEM((ROWS_PER_TILE, CHUNK), jnp.int32),
        pltpu.VMEM((ROWS_PER_TILE, CHUNK), jnp.int32),
        pltpu.VMEM((CHUNK, F), jnp.float32),
        pltpu.VMEM((STRIPE, F), jnp.float32),
        pltpu.VMEM_SHARED((NP, F), jnp.float32),
        pltpu.SemaphoreType.DMA,
    ],
)
def _edge_kernel(src_hbm, dst_hbm, h_hbm, out_hbm,
                 src_v, dst_v, rows_v, zb_v, acc_sh, sem):
    c = lax.axis_index("c")
    s = lax.axis_index("s")
    slab = c * NS + s
    for i in range(CHUNK):
        rows_v[i] = jnp.zeros((F,), jnp.float32)
    for k in range(STRIPE // CHUNK):
        pltpu.sync_copy(rows_v, acc_sh.at[pl.ds(s * STRIPE + k * CHUNK, CHUNK)])
    plsc.subcore_barrier()
    pltpu.sync_copy(src_hbm.at[slab], src_v)
    pltpu.sync_copy(dst_hbm.at[slab], dst_v)

    def body(j, carry):
        pltpu.async_copy(h_hbm.at[src_v.at[j]], rows_v, sem).wait()
        pltpu.sync_copy(rows_v, acc_sh.at[dst_v.at[j]], add=True)
        return carry

    lax.fori_loop(0, ROWS_PER_TILE, body, 0)
    plsc.subcore_barrier()
    pltpu.sync_copy(acc_sh.at[pl.ds(s * STRIPE, STRIPE)], zb_v)
    pltpu.sync_copy(zb_v, out_hbm.at[c, pl.ds(s * STRIPE, STRIPE)])


def _tc1_body(x_ref, w_ref, dg_ref, dinv_ref, h_ref):
    deg = dg_ref[:, 0:1] + dg_ref[:, 1:2] + 1.0
    dinv = lax.rsqrt(deg)
    dinv_ref[...] = dinv
    h = jnp.dot(x_ref[...], w_ref[...], preferred_element_type=jnp.float32)
    h_ref[...] = h * dinv


def _tc2_body(a_ref, h_ref, dinv_ref, b_ref, w_ref, out_ref):
    dinv = dinv_ref[...]
    s1 = dinv * (a_ref[0] + a_ref[1] + h_ref[...]) + b_ref[...]
    z1 = jnp.maximum(s1, 0.0)
    out_ref[...] = jnp.dot(z1, w_ref[...],
                           preferred_element_type=jnp.float32) * dinv


def _tc3_body(a_ref, h_ref, dinv_ref, b_ref, out_ref):
    dinv = dinv_ref[...]
    s2 = dinv * (a_ref[0] + a_ref[1] + h_ref[...]) + b_ref[...]
    logits = s2[:N, 0:3]
    m = jnp.max(logits, axis=1, keepdims=True)
    e = jnp.exp(logits - m)
    se = jnp.sum(e, axis=1, keepdims=True)
    out_ref[...] = (logits - m) - jnp.log(se)


def kernel(x, edge_index, W1, b1, W2, b2):
    src = edge_index[0].astype(jnp.int32)
    dst = edge_index[1].astype(jnp.int32)
    pad = jnp.full((EP - E,), N, jnp.int32)
    src3 = jnp.concatenate([src, pad]).reshape(NW, ROWS_PER_TILE, CHUNK)
    dst3 = jnp.concatenate([dst, pad]).reshape(NW, ROWS_PER_TILE, CHUNK)
    xp = jnp.pad(x, ((0, NP - N), (0, 0)))
    w2p = jnp.pad(W2, ((0, 0), (0, F - W2.shape[1])))
    b1r = b1.reshape(1, F)
    b2r = jnp.pad(b2, (0, F - b2.shape[0])).reshape(1, F)

    deg_parts = _deg_kernel(dst3)
    deg_t = deg_parts.T

    dinv, h1p = pl.pallas_call(
        _tc1_body,
        out_shape=(jax.ShapeDtypeStruct((NP, 1), jnp.float32),
                   jax.ShapeDtypeStruct((NP, F), jnp.float32)),
    )(xp, W1, deg_t)

    acc1 = _edge_kernel(src3, dst3, h1p)

    h2p = pl.pallas_call(
        _tc2_body,
        out_shape=jax.ShapeDtypeStruct((NP, F), jnp.float32),
    )(acc1, h1p, dinv, b1r, w2p)

    acc2 = _edge_kernel(src3, dst3, h2p)

    out = pl.pallas_call(
        _tc3_body,
        out_shape=jax.ShapeDtypeStruct((N, 3), jnp.float32),
    )(acc2, h2p, dinv, b2r)
    return out


# speedup vs baseline: 41.0059x; 1.1540x over previous
import functools

import jax
import jax.numpy as jnp
from jax import lax
from jax.experimental import pallas as pl
from jax.experimental.pallas import tpu as pltpu
from jax.experimental.pallas import tpu_sc as plsc

N = 10000
NP = 10240
E = 320000
NC = 2
NS = 16
NW = NC * NS
CHUNK = 128
NBUF = 4
ROWS_PER_TILE = 80
EP = NW * ROWS_PER_TILE * CHUNK
STRIPE = NP // NS
F = 16

_mesh = plsc.VectorSubcoreMesh(core_axis_name="c", subcore_axis_name="s")
_sc_params = pltpu.CompilerParams(use_tc_tiling_on_sc=False)


@functools.partial(
    pl.kernel,
    mesh=_mesh,
    compiler_params=_sc_params,
    out_type=jax.ShapeDtypeStruct((NC, NP), jnp.float32),
    scratch_types=[
        pltpu.VMEM((ROWS_PER_TILE, CHUNK), jnp.int32),
        pltpu.VMEM((CHUNK,), jnp.float32),
        pltpu.VMEM((STRIPE,), jnp.float32),
        pltpu.VMEM_SHARED((NP,), jnp.float32),
        pltpu.SemaphoreType.DMA,
    ],
)
def _deg_kernel(dst_hbm, out_hbm, dst_v, ones_v, zb_v, deg_sh, sem):
    c = lax.axis_index("c")
    s = lax.axis_index("s")
    slab = c * NS + s
    for i in range(CHUNK // 16):
        ones_v[pl.ds(i * 16, 16)] = jnp.full((16,), 1.0, jnp.float32)
    for i in range(STRIPE // 16):
        zb_v[pl.ds(i * 16, 16)] = jnp.zeros((16,), jnp.float32)
    pltpu.sync_copy(zb_v, deg_sh.at[pl.ds(s * STRIPE, STRIPE)])
    plsc.subcore_barrier()
    pltpu.sync_copy(dst_hbm.at[slab], dst_v)

    def body(j, carry):
        pltpu.sync_copy(ones_v, deg_sh.at[dst_v.at[j]], add=True)
        return carry

    lax.fori_loop(0, ROWS_PER_TILE, body, 0)
    plsc.subcore_barrier()
    pltpu.sync_copy(deg_sh.at[pl.ds(s * STRIPE, STRIPE)], zb_v)
    pltpu.sync_copy(zb_v, out_hbm.at[c, pl.ds(s * STRIPE, STRIPE)])


@functools.partial(
    pl.kernel,
    mesh=_mesh,
    compiler_params=_sc_params,
    out_type=jax.ShapeDtypeStruct((NC, NP, F), jnp.float32),
    scratch_types=[
        pltpu.VMEM((ROWS_PER_TILE, CHUNK), jnp.int32),
        pltpu.VMEM((ROWS_PER_TILE, CHUNK), jnp.int32),
    ] + [pltpu.VMEM((CHUNK, F), jnp.float32) for _ in range(NBUF)] + [
        pltpu.VMEM((STRIPE, F), jnp.float32),
    ] + [pltpu.SemaphoreType.DMA for _ in range(2 * NBUF)] + [
        pltpu.VMEM_SHARED((NP, F), jnp.float32),
    ],
)
def _edge_kernel(src_hbm, dst_hbm, h_hbm, out_hbm,
                 src_v, dst_v, r0, r1, r2, r3, zb_v,
                 g0, g1, g2, g3, s0, s1, s2, s3, acc_sh):
    c = lax.axis_index("c")
    s = lax.axis_index("s")
    slab = c * NS + s
    rows = [r0, r1, r2, r3]
    gsem = [g0, g1, g2, g3]
    ssem = [s0, s1, s2, s3]
    for i in range(CHUNK):
        r0[i] = jnp.zeros((F,), jnp.float32)
    for k in range(STRIPE // CHUNK):
        pltpu.sync_copy(r0, acc_sh.at[pl.ds(s * STRIPE + k * CHUNK, CHUNK)])
    plsc.subcore_barrier()
    pltpu.sync_copy(src_hbm.at[slab], src_v)
    pltpu.sync_copy(dst_hbm.at[slab], dst_v)

    for b in range(NBUF):
        pltpu.async_copy(h_hbm.at[src_v.at[b]], rows[b], gsem[b])

    def body(g, carry):
        for b in range(NBUF):
            j = g * NBUF + b
            pltpu.make_async_copy(h_hbm.at[src_v.at[j]], rows[b],
                                  gsem[b]).wait()
            pltpu.async_copy(rows[b], acc_sh.at[dst_v.at[j]], ssem[b],
                             add=True)
        for b in range(NBUF):
            jn = (g + 1) * NBUF + b
            pltpu.make_async_copy(rows[b], acc_sh.at[dst_v.at[jn]],
                                  ssem[b]).wait()
            pltpu.async_copy(h_hbm.at[src_v.at[jn]], rows[b], gsem[b])
        return carry

    lax.fori_loop(0, ROWS_PER_TILE // NBUF - 1, body, 0)
    for b in range(NBUF):
        j = ROWS_PER_TILE - NBUF + b
        pltpu.make_async_copy(h_hbm.at[src_v.at[j]], rows[b], gsem[b]).wait()
        pltpu.async_copy(rows[b], acc_sh.at[dst_v.at[j]], ssem[b], add=True)
    for b in range(NBUF):
        j = ROWS_PER_TILE - NBUF + b
        pltpu.make_async_copy(rows[b], acc_sh.at[dst_v.at[j]], ssem[b]).wait()
    plsc.subcore_barrier()
    pltpu.sync_copy(acc_sh.at[pl.ds(s * STRIPE, STRIPE)], zb_v)
    pltpu.sync_copy(zb_v, out_hbm.at[c, pl.ds(s * STRIPE, STRIPE)])


def _tc1_body(x_ref, w_ref, dg_ref, dinv_ref, h_ref):
    deg = dg_ref[:, 0:1] + dg_ref[:, 1:2] + 1.0
    dinv = lax.rsqrt(deg)
    dinv_ref[...] = dinv
    h = jnp.dot(x_ref[...], w_ref[...], preferred_element_type=jnp.float32)
    h_ref[...] = h * dinv


def _tc2_body(a_ref, h_ref, dinv_ref, b_ref, w_ref, out_ref):
    dinv = dinv_ref[...]
    s1 = dinv * (a_ref[0] + a_ref[1] + h_ref[...]) + b_ref[...]
    z1 = jnp.maximum(s1, 0.0)
    out_ref[...] = jnp.dot(z1, w_ref[...],
                           preferred_element_type=jnp.float32) * dinv


def _tc3_body(a_ref, h_ref, dinv_ref, b_ref, out_ref):
    dinv = dinv_ref[...]
    s2 = dinv * (a_ref[0] + a_ref[1] + h_ref[...]) + b_ref[...]
    logits = s2[:N, 0:3]
    m = jnp.max(logits, axis=1, keepdims=True)
    e = jnp.exp(logits - m)
    se = jnp.sum(e, axis=1, keepdims=True)
    out_ref[...] = (logits - m) - jnp.log(se)


def kernel(x, edge_index, W1, b1, W2, b2):
    src = edge_index[0].astype(jnp.int32)
    dst = edge_index[1].astype(jnp.int32)
    pad = jnp.full((EP - E,), N, jnp.int32)
    src3 = jnp.concatenate([src, pad]).reshape(NW, ROWS_PER_TILE, CHUNK)
    dst3 = jnp.concatenate([dst, pad]).reshape(NW, ROWS_PER_TILE, CHUNK)
    xp = jnp.pad(x, ((0, NP - N), (0, 0)))
    w2p = jnp.pad(W2, ((0, 0), (0, F - W2.shape[1])))
    b1r = b1.reshape(1, F)
    b2r = jnp.pad(b2, (0, F - b2.shape[0])).reshape(1, F)

    deg_parts = _deg_kernel(dst3)
    deg_t = deg_parts.T

    dinv, h1p = pl.pallas_call(
        _tc1_body,
        out_shape=(jax.ShapeDtypeStruct((NP, 1), jnp.float32),
                   jax.ShapeDtypeStruct((NP, F), jnp.float32)),
    )(xp, W1, deg_t)

    acc1 = _edge_kernel(src3, dst3, h1p)

    h2p = pl.pallas_call(
        _tc2_body,
        out_shape=jax.ShapeDtypeStruct((NP, F), jnp.float32),
    )(acc1, h1p, dinv, b1r, w2p)

    acc2 = _edge_kernel(src3, dst3, h2p)

    out = pl.pallas_call(
        _tc3_body,
        out_shape=jax.ShapeDtypeStruct((N, 3), jnp.float32),
    )(acc2, h2p, dinv, b2r)
    return out


# speedup vs baseline: 56.1925x; 1.3704x over previous
import functools

import jax
import jax.numpy as jnp
from jax import lax
from jax.experimental import pallas as pl
from jax.experimental.pallas import tpu as pltpu
from jax.experimental.pallas import tpu_sc as plsc

N = 10000
NP = 10240
E = 320000
NC = 2
NS = 16
NW = NC * NS
CHUNK = 128
NBUF = 4
ROWS_PER_TILE = 80
EP = NW * ROWS_PER_TILE * CHUNK
STRIPE = NP // NS
F = 16

_mesh = plsc.VectorSubcoreMesh(core_axis_name="c", subcore_axis_name="s")
_sc_params = pltpu.CompilerParams(use_tc_tiling_on_sc=False)


@functools.partial(
    pl.kernel,
    mesh=_mesh,
    compiler_params=_sc_params,
    out_type=jax.ShapeDtypeStruct((NC, NP), jnp.float32),
    scratch_types=[
        pltpu.VMEM((ROWS_PER_TILE, CHUNK), jnp.int32),
        pltpu.VMEM((CHUNK,), jnp.float32),
        pltpu.VMEM((STRIPE,), jnp.float32),
        pltpu.VMEM_SHARED((NP,), jnp.float32),
        pltpu.SemaphoreType.DMA,
    ],
)
def _deg_kernel(dst_hbm, out_hbm, dst_v, ones_v, zb_v, deg_sh, sem):
    c = lax.axis_index("c")
    s = lax.axis_index("s")
    slab = c * NS + s
    for i in range(CHUNK // 16):
        ones_v[pl.ds(i * 16, 16)] = jnp.full((16,), 1.0, jnp.float32)
    for i in range(STRIPE // 16):
        zb_v[pl.ds(i * 16, 16)] = jnp.zeros((16,), jnp.float32)
    pltpu.sync_copy(zb_v, deg_sh.at[pl.ds(s * STRIPE, STRIPE)])
    plsc.subcore_barrier()
    pltpu.sync_copy(dst_hbm.at[slab], dst_v)

    def body(j, carry):
        pltpu.sync_copy(ones_v, deg_sh.at[dst_v.at[j]], add=True)
        return carry

    lax.fori_loop(0, ROWS_PER_TILE, body, 0)
    plsc.subcore_barrier()
    pltpu.sync_copy(deg_sh.at[pl.ds(s * STRIPE, STRIPE)], zb_v)
    pltpu.sync_copy(zb_v, out_hbm.at[c, pl.ds(s * STRIPE, STRIPE)])


@functools.partial(
    pl.kernel,
    mesh=_mesh,
    compiler_params=_sc_params,
    out_type=jax.ShapeDtypeStruct((NC, NP, F), jnp.float32),
    scratch_types=[
        pltpu.VMEM((ROWS_PER_TILE, CHUNK), jnp.int32),
        pltpu.VMEM((ROWS_PER_TILE, CHUNK), jnp.int32),
    ] + [pltpu.VMEM((CHUNK, F), jnp.float32) for _ in range(NBUF)] + [
        pltpu.VMEM((STRIPE, F), jnp.float32),
    ] + [pltpu.SemaphoreType.DMA for _ in range(2 * NBUF)] + [
        pltpu.VMEM_SHARED((NP, F), jnp.float32),
        pltpu.VMEM_SHARED((NP, F), jnp.float32),
    ],
)
def _edge_kernel(src_hbm, dst_hbm, h_hbm, out_hbm,
                 src_v, dst_v, r0, r1, r2, r3, zb_v,
                 g0, g1, g2, g3, s0, s1, s2, s3, acc_sh, h_sh):
    c = lax.axis_index("c")
    s = lax.axis_index("s")
    slab = c * NS + s
    rows = [r0, r1, r2, r3]
    gsem = [g0, g1, g2, g3]
    ssem = [s0, s1, s2, s3]
    for i in range(CHUNK):
        r0[i] = jnp.zeros((F,), jnp.float32)
    for k in range(STRIPE // CHUNK):
        pltpu.sync_copy(r0, acc_sh.at[pl.ds(s * STRIPE + k * CHUNK, CHUNK)])
    pltpu.sync_copy(h_hbm.at[pl.ds(s * STRIPE, STRIPE)], zb_v)
    pltpu.sync_copy(zb_v, h_sh.at[pl.ds(s * STRIPE, STRIPE)])
    plsc.subcore_barrier()
    pltpu.sync_copy(src_hbm.at[slab], src_v)
    pltpu.sync_copy(dst_hbm.at[slab], dst_v)

    for b in range(NBUF):
        pltpu.async_copy(h_sh.at[src_v.at[b]], rows[b], gsem[b])

    def body(g, carry):
        for b in range(NBUF):
            j = g * NBUF + b
            pltpu.make_async_copy(h_sh.at[src_v.at[j]], rows[b],
                                  gsem[b]).wait()
            pltpu.async_copy(rows[b], acc_sh.at[dst_v.at[j]], ssem[b],
                             add=True)
        for b in range(NBUF):
            jn = (g + 1) * NBUF + b
            pltpu.make_async_copy(rows[b], acc_sh.at[dst_v.at[jn]],
                                  ssem[b]).wait()
            pltpu.async_copy(h_sh.at[src_v.at[jn]], rows[b], gsem[b])
        return carry

    lax.fori_loop(0, ROWS_PER_TILE // NBUF - 1, body, 0)
    for b in range(NBUF):
        j = ROWS_PER_TILE - NBUF + b
        pltpu.make_async_copy(h_sh.at[src_v.at[j]], rows[b], gsem[b]).wait()
        pltpu.async_copy(rows[b], acc_sh.at[dst_v.at[j]], ssem[b], add=True)
    for b in range(NBUF):
        j = ROWS_PER_TILE - NBUF + b
        pltpu.make_async_copy(rows[b], acc_sh.at[dst_v.at[j]], ssem[b]).wait()
    plsc.subcore_barrier()
    pltpu.sync_copy(acc_sh.at[pl.ds(s * STRIPE, STRIPE)], zb_v)
    pltpu.sync_copy(zb_v, out_hbm.at[c, pl.ds(s * STRIPE, STRIPE)])


def _tc1_body(x_ref, w_ref, dg_ref, dinv_ref, h_ref):
    deg = dg_ref[:, 0:1] + dg_ref[:, 1:2] + 1.0
    dinv = lax.rsqrt(deg)
    dinv_ref[...] = dinv
    h = jnp.dot(x_ref[...], w_ref[...], preferred_element_type=jnp.float32)
    h_ref[...] = h * dinv


def _tc2_body(a_ref, h_ref, dinv_ref, b_ref, w_ref, out_ref):
    dinv = dinv_ref[...]
    s1 = dinv * (a_ref[0] + a_ref[1] + h_ref[...]) + b_ref[...]
    z1 = jnp.maximum(s1, 0.0)
    out_ref[...] = jnp.dot(z1, w_ref[...],
                           preferred_element_type=jnp.float32) * dinv


def _tc3_body(a_ref, h_ref, dinv_ref, b_ref, out_ref):
    dinv = dinv_ref[...]
    s2 = dinv * (a_ref[0] + a_ref[1] + h_ref[...]) + b_ref[...]
    logits = s2[:N, 0:3]
    m = jnp.max(logits, axis=1, keepdims=True)
    e = jnp.exp(logits - m)
    se = jnp.sum(e, axis=1, keepdims=True)
    out_ref[...] = (logits - m) - jnp.log(se)


def kernel(x, edge_index, W1, b1, W2, b2):
    src = edge_index[0].astype(jnp.int32)
    dst = edge_index[1].astype(jnp.int32)
    pad = jnp.full((EP - E,), N, jnp.int32)
    src3 = jnp.concatenate([src, pad]).reshape(NW, ROWS_PER_TILE, CHUNK)
    dst3 = jnp.concatenate([dst, pad]).reshape(NW, ROWS_PER_TILE, CHUNK)
    xp = jnp.pad(x, ((0, NP - N), (0, 0)))
    w2p = jnp.pad(W2, ((0, 0), (0, F - W2.shape[1])))
    b1r = b1.reshape(1, F)
    b2r = jnp.pad(b2, (0, F - b2.shape[0])).reshape(1, F)

    deg_parts = _deg_kernel(dst3)
    deg_t = deg_parts.T

    dinv, h1p = pl.pallas_call(
        _tc1_body,
        out_shape=(jax.ShapeDtypeStruct((NP, 1), jnp.float32),
                   jax.ShapeDtypeStruct((NP, F), jnp.float32)),
    )(xp, W1, deg_t)

    acc1 = _edge_kernel(src3, dst3, h1p)

    h2p = pl.pallas_call(
        _tc2_body,
        out_shape=jax.ShapeDtypeStruct((NP, F), jnp.float32),
    )(acc1, h1p, dinv, b1r, w2p)

    acc2 = _edge_kernel(src3, dst3, h2p)

    out = pl.pallas_call(
        _tc3_body,
        out_shape=jax.ShapeDtypeStruct((N, 3), jnp.float32),
    )(acc2, h2p, dinv, b2r)
    return out


# speedup vs baseline: 72.0810x; 1.2828x over previous
import functools

import numpy as np
import jax
import jax.numpy as jnp
from jax import lax
from jax.experimental import pallas as pl
from jax.experimental.pallas import tpu as pltpu
from jax.experimental.pallas import tpu_sc as plsc

N = 10000
NP = 10240
WR = NP // 8
E = 320000
NC = 2
NS = 16
NW = NC * NS
CHUNK = 128
NBUF = 4
ROWS_PER_TILE = 80
EP = NW * ROWS_PER_TILE * CHUNK
STRIPE = NP // NS
F = 16

_mesh = plsc.VectorSubcoreMesh(core_axis_name="c", subcore_axis_name="s")
_sc_params = pltpu.CompilerParams(use_tc_tiling_on_sc=False)

_S_np = np.zeros((128, 128), np.float32)
for _j in range(8):
    _S_np[16 * _j, 16 * _j:16 * _j + 16] = 1.0


@functools.partial(
    pl.kernel,
    mesh=_mesh,
    compiler_params=_sc_params,
    out_type=jax.ShapeDtypeStruct((NC * NP * F,), jnp.float32),
    scratch_types=[
        pltpu.VMEM((ROWS_PER_TILE, CHUNK), jnp.int32),
        pltpu.VMEM((ROWS_PER_TILE, CHUNK), jnp.int32),
        pltpu.VMEM((CHUNK,), jnp.float32),
        pltpu.VMEM((NP,), jnp.float32),
        pltpu.VMEM_SHARED((NP * F,), jnp.float32),
    ],
)
def _deg_kernel(dst_hbm, out_hbm, dst_v, didx_v, ones_v, rb_v, deg_sh):
    c = lax.axis_index("c")
    s = lax.axis_index("s")
    slab = c * NS + s
    for i in range(CHUNK // 16):
        ones_v[pl.ds(i * 16, 16)] = jnp.full((16,), 1.0, jnp.float32)

    def zf(i, carry):
        rb_v[pl.ds(i * 16, 16)] = jnp.zeros((16,), jnp.float32)
        return carry

    lax.fori_loop(0, NP // 16, zf, 0)
    pltpu.sync_copy(rb_v, deg_sh.at[pl.ds(s * NP, NP)])
    pltpu.sync_copy(dst_hbm.at[slab], dst_v)

    def shl(j, carry):
        for k in range(CHUNK // 16):
            v = dst_v[j, pl.ds(16 * k, 16)]
            didx_v[j, pl.ds(16 * k, 16)] = v * 16
        return carry

    lax.fori_loop(0, ROWS_PER_TILE, shl, 0)
    plsc.subcore_barrier()

    def body(j, carry):
        pltpu.sync_copy(ones_v, deg_sh.at[didx_v.at[j]], add=True)
        return carry

    lax.fori_loop(0, ROWS_PER_TILE, body, 0)
    plsc.subcore_barrier()
    pltpu.sync_copy(deg_sh.at[pl.ds(s * NP, NP)], rb_v)
    pltpu.sync_copy(rb_v, out_hbm.at[pl.ds(c * NP * F + s * NP, NP)])


@functools.partial(
    pl.kernel,
    mesh=_mesh,
    compiler_params=_sc_params,
    out_type=jax.ShapeDtypeStruct((NC, NP, F), jnp.float32),
    scratch_types=[
        pltpu.VMEM((ROWS_PER_TILE, CHUNK), jnp.int32),
        pltpu.VMEM((ROWS_PER_TILE, CHUNK), jnp.int32),
    ] + [pltpu.VMEM((CHUNK, F), jnp.float32) for _ in range(NBUF)] + [
        pltpu.VMEM((STRIPE, F), jnp.float32),
    ] + [pltpu.SemaphoreType.DMA for _ in range(2 * NBUF)] + [
        pltpu.VMEM_SHARED((NP, F), jnp.float32),
        pltpu.VMEM_SHARED((NP, F), jnp.float32),
    ],
)
def _edge_kernel(src_hbm, dst_hbm, h_hbm, out_hbm,
                 src_v, dst_v, r0, r1, r2, r3, zb_v,
                 g0, g1, g2, g3, s0, s1, s2, s3, acc_sh, h_sh):
    c = lax.axis_index("c")
    s = lax.axis_index("s")
    slab = c * NS + s
    rows = [r0, r1, r2, r3]
    gsem = [g0, g1, g2, g3]
    ssem = [s0, s1, s2, s3]
    for i in range(CHUNK):
        r0[i] = jnp.zeros((F,), jnp.float32)
    for k in range(STRIPE // CHUNK):
        pltpu.sync_copy(r0, acc_sh.at[pl.ds(s * STRIPE + k * CHUNK, CHUNK)])
    pltpu.sync_copy(h_hbm.at[pl.ds(s * STRIPE, STRIPE)], zb_v)
    pltpu.sync_copy(zb_v, h_sh.at[pl.ds(s * STRIPE, STRIPE)])
    plsc.subcore_barrier()
    pltpu.sync_copy(src_hbm.at[slab], src_v)
    pltpu.sync_copy(dst_hbm.at[slab], dst_v)

    for b in range(NBUF):
        pltpu.async_copy(h_sh.at[src_v.at[b]], rows[b], gsem[b])

    def body(g, carry):
        for b in range(NBUF):
            j = g * NBUF + b
            pltpu.make_async_copy(h_sh.at[src_v.at[j]], rows[b],
                                  gsem[b]).wait()
            pltpu.async_copy(rows[b], acc_sh.at[dst_v.at[j]], ssem[b],
                             add=True)
        for b in range(NBUF):
            jn = (g + 1) * NBUF + b
            pltpu.make_async_copy(rows[b], acc_sh.at[dst_v.at[jn]],
                                  ssem[b]).wait()
            pltpu.async_copy(h_sh.at[src_v.at[jn]], rows[b], gsem[b])
        return carry

    lax.fori_loop(0, ROWS_PER_TILE // NBUF - 1, body, 0)
    for b in range(NBUF):
        j = ROWS_PER_TILE - NBUF + b
        pltpu.make_async_copy(h_sh.at[src_v.at[j]], rows[b], gsem[b]).wait()
        pltpu.async_copy(rows[b], acc_sh.at[dst_v.at[j]], ssem[b], add=True)
    for b in range(NBUF):
        j = ROWS_PER_TILE - NBUF + b
        pltpu.make_async_copy(rows[b], acc_sh.at[dst_v.at[j]], ssem[b]).wait()
    plsc.subcore_barrier()
    pltpu.sync_copy(acc_sh.at[pl.ds(s * STRIPE, STRIPE)], zb_v)
    pltpu.sync_copy(zb_v, out_hbm.at[c, pl.ds(s * STRIPE, STRIPE)])


def _tc1_body(x_ref, w_ref, dg_ref, s_ref, dinv_ref, h_ref):
    dgv = dg_ref[...].reshape(2 * WR, 128)
    deg = jnp.dot(dgv[:WR] + dgv[WR:], s_ref[...],
                  preferred_element_type=jnp.float32) + 1.0
    dinv = lax.rsqrt(deg)
    dinv_ref[...] = dinv
    h = jnp.dot(x_ref[...], w_ref[...], preferred_element_type=jnp.float32)
    h_ref[...] = (h * dinv).reshape(NP * F)


def _tc2_body(a_ref, h_ref, dinv_ref, b_ref, w_ref, out_ref):
    dinv = dinv_ref[...]
    av = a_ref[...].reshape(2 * WR, 128)
    hv = h_ref[...].reshape(WR, 128)
    s1 = dinv * (av[:WR] + av[WR:] + hv) + b_ref[...]
    z1 = jnp.maximum(s1, 0.0)
    h2 = jnp.dot(z1, w_ref[...], preferred_element_type=jnp.float32) * dinv
    out_ref[...] = h2.reshape(NP * F)


def _tc3_body(a_ref, h_ref, dinv_ref, b_ref, out_ref):
    dinv = dinv_ref[...]
    av = a_ref[...].reshape(2 * WR, 128)
    hv = h_ref[...].reshape(WR, 128)
    s2 = dinv * (av[:WR] + av[WR:] + hv) + b_ref[...]
    lane = lax.broadcasted_iota(jnp.int32, (WR, 128), 1)
    is0 = (lane % F) == 0
    m = jnp.maximum(jnp.maximum(s2, pltpu.roll(s2, 127, 1)),
                    pltpu.roll(s2, 126, 1))
    m0 = jnp.where(is0, m, 0.0)
    msp = m0 + pltpu.roll(m0, 1, 1) + pltpu.roll(m0, 2, 1)
    e = jnp.exp(s2 - msp)
    se = e + pltpu.roll(e, 127, 1) + pltpu.roll(e, 126, 1)
    se0 = jnp.where(is0, jnp.log(se), 0.0)
    lsp = se0 + pltpu.roll(se0, 1, 1) + pltpu.roll(se0, 2, 1)
    out_ref[...] = ((s2 - msp) - lsp).reshape(NP * F)


def kernel(x, edge_index, W1, b1, W2, b2):
    src = edge_index[0].astype(jnp.int32)
    dst = edge_index[1].astype(jnp.int32)
    pad = jnp.full((EP - E,), N, jnp.int32)
    src3 = jnp.concatenate([src, pad]).reshape(NW, ROWS_PER_TILE, CHUNK)
    dst3 = jnp.concatenate([dst, pad]).reshape(NW, ROWS_PER_TILE, CHUNK)
    xw = jnp.pad(x, ((0, NP - N), (0, 0))).reshape(WR, 8 * 128)
    eye8 = jnp.eye(8, dtype=jnp.float32)
    w1blk = jnp.kron(eye8, W1)
    w2p = jnp.pad(W2, ((0, 0), (0, F - W2.shape[1])))
    w2blk = jnp.kron(eye8, w2p)
    b1w = jnp.tile(b1, 8).reshape(1, 128)
    b2w = jnp.tile(jnp.pad(b2, (0, F - b2.shape[0])), 8).reshape(1, 128)
    smat = jnp.asarray(_S_np)

    degw = _deg_kernel(dst3)

    dinv, h1p = pl.pallas_call(
        _tc1_body,
        out_shape=(jax.ShapeDtypeStruct((WR, 128), jnp.float32),
                   jax.ShapeDtypeStruct((NP * F,), jnp.float32)),
    )(xw, w1blk, degw, smat)

    acc1 = _edge_kernel(src3, dst3, h1p.reshape(NP, F))

    h2p = pl.pallas_call(
        _tc2_body,
        out_shape=jax.ShapeDtypeStruct((NP * F,), jnp.float32),
    )(acc1.reshape(NC * NP * F), h1p, dinv, b1w, w2blk)

    acc2 = _edge_kernel(src3, dst3, h2p.reshape(NP, F))

    outw = pl.pallas_call(
        _tc3_body,
        out_shape=jax.ShapeDtypeStruct((NP * F,), jnp.float32),
    )(acc2.reshape(NC * NP * F), h2p, dinv, b2w)
    return outw.reshape(NP, F)[:N, :3]


# speedup vs baseline: 72.3525x; 1.0038x over previous
import functools

import numpy as np
import jax
import jax.numpy as jnp
from jax import lax
from jax.experimental import pallas as pl
from jax.experimental.pallas import tpu as pltpu
from jax.experimental.pallas import tpu_sc as plsc

N = 10000
NP = 10240
WR = NP // 8
E = 320000
NC = 2
NS = 16
NW = NC * NS
CHUNK = 128
NBUF = 4
ROWS_PER_TILE = 80
SCHUNK = 256
SROWS = ROWS_PER_TILE * CHUNK // SCHUNK
EP = NW * ROWS_PER_TILE * CHUNK
STRIPE = NP // NS
F = 16

_mesh = plsc.VectorSubcoreMesh(core_axis_name="c", subcore_axis_name="s")
_sc_params = pltpu.CompilerParams(use_tc_tiling_on_sc=False)

_S_np = np.zeros((128, 128), np.float32)
for _j in range(8):
    _S_np[16 * _j, 16 * _j:16 * _j + 16] = 1.0


@functools.partial(
    pl.kernel,
    mesh=_mesh,
    compiler_params=_sc_params,
    out_type=jax.ShapeDtypeStruct((NC * NP * F,), jnp.float32),
    scratch_types=[
        pltpu.VMEM((ROWS_PER_TILE, CHUNK), jnp.int32),
        pltpu.VMEM((ROWS_PER_TILE, CHUNK), jnp.int32),
        pltpu.VMEM((CHUNK,), jnp.float32),
        pltpu.VMEM((NP,), jnp.float32),
        pltpu.VMEM_SHARED((NP * F,), jnp.float32),
    ],
)
def _deg_kernel(dst_hbm, out_hbm, dst_v, didx_v, ones_v, rb_v, deg_sh):
    c = lax.axis_index("c")
    s = lax.axis_index("s")
    slab = c * NS + s
    for i in range(CHUNK // 16):
        ones_v[pl.ds(i * 16, 16)] = jnp.full((16,), 1.0, jnp.float32)

    def zf(i, carry):
        rb_v[pl.ds(i * 16, 16)] = jnp.zeros((16,), jnp.float32)
        return carry

    lax.fori_loop(0, NP // 16, zf, 0)
    pltpu.sync_copy(rb_v, deg_sh.at[pl.ds(s * NP, NP)])
    pltpu.sync_copy(dst_hbm.at[slab], dst_v)

    def shl(j, carry):
        for k in range(CHUNK // 16):
            v = dst_v[j, pl.ds(16 * k, 16)]
            didx_v[j, pl.ds(16 * k, 16)] = v * 16
        return carry

    lax.fori_loop(0, ROWS_PER_TILE, shl, 0)
    plsc.subcore_barrier()

    def body(j, carry):
        pltpu.sync_copy(ones_v, deg_sh.at[didx_v.at[j]], add=True)
        return carry

    lax.fori_loop(0, ROWS_PER_TILE, body, 0)
    plsc.subcore_barrier()
    pltpu.sync_copy(deg_sh.at[pl.ds(s * NP, NP)], rb_v)
    pltpu.sync_copy(rb_v, out_hbm.at[pl.ds(c * NP * F + s * NP, NP)])


@functools.partial(
    pl.kernel,
    mesh=_mesh,
    compiler_params=_sc_params,
    out_type=jax.ShapeDtypeStruct((NC, NP, F), jnp.float32),
    scratch_types=[
        pltpu.VMEM((SROWS, SCHUNK), jnp.int32),
        pltpu.VMEM((SROWS, SCHUNK), jnp.int32),
    ] + [pltpu.VMEM((SCHUNK, F), jnp.float32) for _ in range(NBUF)] + [
        pltpu.VMEM((STRIPE, F), jnp.float32),
    ] + [pltpu.SemaphoreType.DMA for _ in range(2 * NBUF)] + [
        pltpu.VMEM_SHARED((NP, F), jnp.float32),
        pltpu.VMEM_SHARED((NP, F), jnp.float32),
    ],
)
def _edge_kernel(src_hbm, dst_hbm, h_hbm, out_hbm,
                 src_v, dst_v, r0, r1, r2, r3, zb_v,
                 g0, g1, g2, g3, s0, s1, s2, s3, acc_sh, h_sh):
    c = lax.axis_index("c")
    s = lax.axis_index("s")
    slab = c * NS + s
    rows = [r0, r1, r2, r3]
    gsem = [g0, g1, g2, g3]
    ssem = [s0, s1, s2, s3]
    pltpu.sync_copy(h_hbm.at[pl.ds(s * STRIPE, STRIPE)], zb_v)
    pltpu.sync_copy(zb_v, h_sh.at[pl.ds(s * STRIPE, STRIPE)])

    @pl.when(c == 0)
    def _():
        pltpu.sync_copy(zb_v, acc_sh.at[pl.ds(s * STRIPE, STRIPE)])

    @pl.when(c != 0)
    def _():
        for i in range(CHUNK):
            r0[i] = jnp.zeros((F,), jnp.float32)
        for k in range(STRIPE // CHUNK):
            pltpu.sync_copy(r0.at[pl.ds(0, CHUNK)],
                            acc_sh.at[pl.ds(s * STRIPE + k * CHUNK, CHUNK)])

    plsc.subcore_barrier()
    pltpu.sync_copy(src_hbm.at[slab], src_v)
    pltpu.sync_copy(dst_hbm.at[slab], dst_v)
    nstream = SROWS

    def gref(j):
        return h_sh.at[src_v.at[j]]

    def sref(j):
        return acc_sh.at[dst_v.at[j]]

    for b in range(NBUF):
        pltpu.async_copy(gref(b), rows[b], gsem[b])

    def body(g, carry):
        for b in range(NBUF):
            j = g * NBUF + b
            pltpu.make_async_copy(gref(j), rows[b], gsem[b]).wait()
            pltpu.async_copy(rows[b], sref(j), ssem[b], add=True)
        for b in range(NBUF):
            jn = (g + 1) * NBUF + b
            pltpu.make_async_copy(rows[b], sref(jn), ssem[b]).wait()
            pltpu.async_copy(gref(jn), rows[b], gsem[b])
        return carry

    lax.fori_loop(0, nstream // NBUF - 1, body, 0)
    for b in range(NBUF):
        j = nstream - NBUF + b
        pltpu.make_async_copy(gref(j), rows[b], gsem[b]).wait()
        pltpu.async_copy(rows[b], sref(j), ssem[b], add=True)
    for b in range(NBUF):
        j = nstream - NBUF + b
        pltpu.make_async_copy(rows[b], sref(j), ssem[b]).wait()
    plsc.subcore_barrier()
    pltpu.sync_copy(acc_sh.at[pl.ds(s * STRIPE, STRIPE)], zb_v)
    pltpu.sync_copy(zb_v, out_hbm.at[c, pl.ds(s * STRIPE, STRIPE)])


def _tc1_body(x_ref, w_ref, dg_ref, s_ref, dinv_ref, h_ref):
    dgv = dg_ref[...].reshape(2 * WR, 128)
    deg = jnp.dot(dgv[:WR] + dgv[WR:], s_ref[...],
                  preferred_element_type=jnp.float32) + 1.0
    dinv = lax.rsqrt(deg)
    dinv_ref[...] = dinv
    h = jnp.dot(x_ref[...], w_ref[...], preferred_element_type=jnp.float32)
    h_ref[...] = (h * dinv).reshape(NP * F)


def _tc2_body(a_ref, dinv_ref, b_ref, w_ref, out_ref):
    dinv = dinv_ref[...]
    av = a_ref[...].reshape(2 * WR, 128)
    s1 = dinv * (av[:WR] + av[WR:]) + b_ref[...]
    z1 = jnp.maximum(s1, 0.0)
    h2 = jnp.dot(z1, w_ref[...], preferred_element_type=jnp.float32) * dinv
    out_ref[...] = h2.reshape(NP * F)


def _tc3_body(a_ref, dinv_ref, b_ref, out_ref):
    dinv = dinv_ref[...]
    av = a_ref[...].reshape(2 * WR, 128)
    s2 = dinv * (av[:WR] + av[WR:]) + b_ref[...]
    lane = lax.broadcasted_iota(jnp.int32, (WR, 128), 1)
    is0 = (lane % F) == 0
    m = jnp.maximum(jnp.maximum(s2, pltpu.roll(s2, 127, 1)),
                    pltpu.roll(s2, 126, 1))
    m0 = jnp.where(is0, m, 0.0)
    msp = m0 + pltpu.roll(m0, 1, 1) + pltpu.roll(m0, 2, 1)
    e = jnp.exp(s2 - msp)
    se = e + pltpu.roll(e, 127, 1) + pltpu.roll(e, 126, 1)
    se0 = jnp.where(is0, jnp.log(se), 0.0)
    lsp = se0 + pltpu.roll(se0, 1, 1) + pltpu.roll(se0, 2, 1)
    out_ref[...] = ((s2 - msp) - lsp).reshape(NP * F)


def kernel(x, edge_index, W1, b1, W2, b2):
    src = edge_index[0].astype(jnp.int32)
    dst = edge_index[1].astype(jnp.int32)
    pad = jnp.full((EP - E,), N, jnp.int32)
    src3 = jnp.concatenate([src, pad]).reshape(NW, ROWS_PER_TILE, CHUNK)
    dst3 = jnp.concatenate([dst, pad]).reshape(NW, ROWS_PER_TILE, CHUNK)
    xw = jnp.pad(x, ((0, NP - N), (0, 0))).reshape(WR, 8 * 128)
    eye8 = jnp.eye(8, dtype=jnp.float32)
    w1blk = jnp.kron(eye8, W1)
    w2p = jnp.pad(W2, ((0, 0), (0, F - W2.shape[1])))
    w2blk = jnp.kron(eye8, w2p)
    b1w = jnp.tile(b1, 8).reshape(1, 128)
    b2w = jnp.tile(jnp.pad(b2, (0, F - b2.shape[0])), 8).reshape(1, 128)
    smat = jnp.asarray(_S_np)

    degw = _deg_kernel(dst3)

    dinv, h1p = pl.pallas_call(
        _tc1_body,
        out_shape=(jax.ShapeDtypeStruct((WR, 128), jnp.float32),
                   jax.ShapeDtypeStruct((NP * F,), jnp.float32)),
    )(xw, w1blk, degw, smat)

    srcw = src3.reshape(NW, SROWS, SCHUNK)
    dstw = dst3.reshape(NW, SROWS, SCHUNK)
    acc1 = _edge_kernel(srcw, dstw, h1p.reshape(NP, F))

    h2p = pl.pallas_call(
        _tc2_body,
        out_shape=jax.ShapeDtypeStruct((NP * F,), jnp.float32),
    )(acc1.reshape(NC * NP * F), dinv, b1w, w2blk)

    acc2 = _edge_kernel(srcw, dstw, h2p.reshape(NP, F))

    outw = pl.pallas_call(
        _tc3_body,
        out_shape=jax.ShapeDtypeStruct((NP * F,), jnp.float32),
    )(acc2.reshape(NC * NP * F), dinv, b2w)
    return outw.reshape(NP, F)[:N, :3]


# speedup vs baseline: 81.8400x; 1.1311x over previous
import functools

import numpy as np
import jax
import jax.numpy as jnp
from jax import lax
from jax.experimental import pallas as pl
from jax.experimental.pallas import tpu as pltpu
from jax.experimental.pallas import tpu_sc as plsc

N = 10000
NP = 10240
WR = NP // 8
E = 320000
NC = 2
NS = 16
NW = NC * NS
CHUNK = 128
NBUF = 4
ROWS_PER_TILE = 80
SCHUNK = 256
SROWS = ROWS_PER_TILE * CHUNK // SCHUNK
EP = NW * ROWS_PER_TILE * CHUNK
STRIPE = NP // NS
F = 16

_mesh = plsc.VectorSubcoreMesh(core_axis_name="c", subcore_axis_name="s")
_sc_params = pltpu.CompilerParams(use_tc_tiling_on_sc=False)

_S_np = np.zeros((128, 128), np.float32)
for _j in range(8):
    _S_np[16 * _j, 16 * _j:16 * _j + 16] = 1.0


EPT = E // NW
NPAD = SROWS * SCHUNK - EPT


@functools.partial(
    pl.kernel,
    mesh=_mesh,
    compiler_params=_sc_params,
    out_type=jax.ShapeDtypeStruct((NC * NP * F,), jnp.float32),
    scratch_types=[
        pltpu.VMEM((SROWS * SCHUNK,), jnp.int32),
        pltpu.VMEM((SROWS * SCHUNK,), jnp.int32),
        pltpu.VMEM((SCHUNK,), jnp.float32),
        pltpu.VMEM((NP,), jnp.float32),
        pltpu.VMEM_SHARED((NP * F,), jnp.float32),
    ],
)
def _deg_kernel(ei_hbm, deg_out, dst_v, didx_v, ones_v, rb_v, deg_sh):
    c = lax.axis_index("c")
    s = lax.axis_index("s")
    slab = c * NS + s
    for i in range(SCHUNK // 16):
        ones_v[pl.ds(i * 16, 16)] = jnp.full((16,), 1.0, jnp.float32)

    def zf(i, carry):
        rb_v[pl.ds(i * 16, 16)] = jnp.zeros((16,), jnp.float32)
        return carry

    lax.fori_loop(0, NP // 16, zf, 0)
    pltpu.sync_copy(rb_v, deg_sh.at[pl.ds(s * NP, NP)])
    pltpu.sync_copy(ei_hbm.at[1, slab], dst_v.at[pl.ds(0, EPT)])

    def shl(j, carry):
        for k in range(SCHUNK // 16):
            v = dst_v[pl.ds(j * SCHUNK + 16 * k, 16)]
            didx_v[pl.ds(j * SCHUNK + 16 * k, 16)] = v * F
        return carry

    lax.fori_loop(0, EPT // SCHUNK, shl, 0)
    for k in range(SCHUNK // 16):
        base = (EPT // SCHUNK) * SCHUNK + 16 * k
        if base < EPT:
            v = dst_v[pl.ds(base, 16)]
            didx_v[pl.ds(base, 16)] = v * F
        else:
            didx_v[pl.ds(base, 16)] = jnp.full((16,), N * F, jnp.int32)
    plsc.subcore_barrier()

    def body(j, carry):
        pltpu.sync_copy(ones_v, deg_sh.at[didx_v.at[pl.ds(j * SCHUNK,
                                                          SCHUNK)]],
                        add=True)
        return carry

    lax.fori_loop(0, SROWS, body, 0)
    plsc.subcore_barrier()
    pltpu.sync_copy(deg_sh.at[pl.ds(s * NP, NP)], rb_v)
    pltpu.sync_copy(rb_v, deg_out.at[pl.ds(c * NP * F + s * NP, NP)])


@functools.partial(
    pl.kernel,
    mesh=_mesh,
    compiler_params=_sc_params,
    out_type=jax.ShapeDtypeStruct((NC, NP, F), jnp.float32),
    scratch_types=[
        pltpu.VMEM((SROWS * SCHUNK,), jnp.int32),
        pltpu.VMEM((SROWS * SCHUNK,), jnp.int32),
    ] + [pltpu.VMEM((SCHUNK, F), jnp.float32) for _ in range(NBUF)] + [
        pltpu.VMEM((STRIPE, F), jnp.float32),
    ] + [pltpu.SemaphoreType.DMA for _ in range(2 * NBUF)] + [
        pltpu.VMEM_SHARED((NP, F), jnp.float32),
        pltpu.VMEM_SHARED((NP, F), jnp.float32),
    ],
)
def _edge_kernel(ei_hbm, h_hbm, out_hbm,
                 src_v, dst_v, r0, r1, r2, r3, zb_v,
                 g0, g1, g2, g3, s0, s1, s2, s3, acc_sh, h_sh):
    c = lax.axis_index("c")
    s = lax.axis_index("s")
    slab = c * NS + s
    rows = [r0, r1, r2, r3]
    gsem = [g0, g1, g2, g3]
    ssem = [s0, s1, s2, s3]
    pltpu.sync_copy(h_hbm.at[pl.ds(s * STRIPE, STRIPE)], zb_v)
    pltpu.sync_copy(zb_v, h_sh.at[pl.ds(s * STRIPE, STRIPE)])

    @pl.when(c == 0)
    def _():
        pltpu.sync_copy(zb_v, acc_sh.at[pl.ds(s * STRIPE, STRIPE)])

    @pl.when(c != 0)
    def _():
        for i in range(CHUNK):
            r0[i] = jnp.zeros((F,), jnp.float32)
        for k in range(STRIPE // CHUNK):
            pltpu.sync_copy(r0.at[pl.ds(0, CHUNK)],
                            acc_sh.at[pl.ds(s * STRIPE + k * CHUNK, CHUNK)])

    plsc.subcore_barrier()
    pltpu.sync_copy(ei_hbm.at[0, slab], src_v.at[pl.ds(0, EPT)])
    pltpu.sync_copy(ei_hbm.at[1, slab], dst_v.at[pl.ds(0, EPT)])
    for k in range(NPAD // 16):
        src_v[pl.ds(EPT + 16 * k, 16)] = jnp.full((16,), N, jnp.int32)
        dst_v[pl.ds(EPT + 16 * k, 16)] = jnp.full((16,), N, jnp.int32)
    nstream = SROWS

    def gref(j):
        return h_sh.at[src_v.at[pl.ds(j * SCHUNK, SCHUNK)]]

    def sref(j):
        return acc_sh.at[dst_v.at[pl.ds(j * SCHUNK, SCHUNK)]]

    for b in range(NBUF):
        pltpu.async_copy(gref(b), rows[b], gsem[b])

    def body(g, carry):
        for b in range(NBUF):
            j = g * NBUF + b
            pltpu.make_async_copy(gref(j), rows[b], gsem[b]).wait()
            pltpu.async_copy(rows[b], sref(j), ssem[b], add=True)
        for b in range(NBUF):
            jn = (g + 1) * NBUF + b
            pltpu.make_async_copy(rows[b], sref(jn), ssem[b]).wait()
            pltpu.async_copy(gref(jn), rows[b], gsem[b])
        return carry

    lax.fori_loop(0, nstream // NBUF - 1, body, 0)
    for b in range(NBUF):
        j = nstream - NBUF + b
        pltpu.make_async_copy(gref(j), rows[b], gsem[b]).wait()
        pltpu.async_copy(rows[b], sref(j), ssem[b], add=True)
    for b in range(NBUF):
        j = nstream - NBUF + b
        pltpu.make_async_copy(rows[b], sref(j), ssem[b]).wait()
    plsc.subcore_barrier()
    pltpu.sync_copy(acc_sh.at[pl.ds(s * STRIPE, STRIPE)], zb_v)
    pltpu.sync_copy(zb_v, out_hbm.at[c, pl.ds(s * STRIPE, STRIPE)])


def _tc1_body(x_ref, w_ref, dg_ref, s_ref, dinv_ref, h_ref):
    dgv = dg_ref[...].reshape(2 * WR, 128)
    deg = jnp.dot(dgv[:WR] + dgv[WR:], s_ref[...],
                  preferred_element_type=jnp.float32) + 1.0
    dinv = lax.rsqrt(deg)
    dinv_ref[...] = dinv
    h = jnp.dot(x_ref[...], w_ref[...], preferred_element_type=jnp.float32)
    h_ref[...] = (h * dinv).reshape(NP * F)


def _tc2_body(a_ref, dinv_ref, b_ref, w_ref, out_ref):
    dinv = dinv_ref[...]
    av = a_ref[...].reshape(2 * WR, 128)
    s1 = dinv * (av[:WR] + av[WR:]) + b_ref[...]
    z1 = jnp.maximum(s1, 0.0)
    h2 = jnp.dot(z1, w_ref[...], preferred_element_type=jnp.float32) * dinv
    out_ref[...] = h2.reshape(NP * F)


def _tc3_body(a_ref, dinv_ref, b_ref, out_ref):
    dinv = dinv_ref[...]
    av = a_ref[...].reshape(2 * WR, 128)
    s2 = dinv * (av[:WR] + av[WR:]) + b_ref[...]
    lane = lax.broadcasted_iota(jnp.int32, (WR, 128), 1)
    is0 = (lane % F) == 0
    m = jnp.maximum(jnp.maximum(s2, pltpu.roll(s2, 127, 1)),
                    pltpu.roll(s2, 126, 1))
    m0 = jnp.where(is0, m, 0.0)
    msp = m0 + pltpu.roll(m0, 1, 1) + pltpu.roll(m0, 2, 1)
    e = jnp.exp(s2 - msp)
    se = e + pltpu.roll(e, 127, 1) + pltpu.roll(e, 126, 1)
    se0 = jnp.where(is0, jnp.log(se), 0.0)
    lsp = se0 + pltpu.roll(se0, 1, 1) + pltpu.roll(se0, 2, 1)
    out_ref[...] = ((s2 - msp) - lsp).reshape(NP * F)


def kernel(x, edge_index, W1, b1, W2, b2):
    ei = edge_index.astype(jnp.int32).reshape(2, NW, EPT)
    xw = jnp.pad(x, ((0, NP - N), (0, 0))).reshape(WR, 8 * 128)
    eye8 = jnp.eye(8, dtype=jnp.float32)
    w1blk = jnp.kron(eye8, W1)
    w2p = jnp.pad(W2, ((0, 0), (0, F - W2.shape[1])))
    w2blk = jnp.kron(eye8, w2p)
    b1w = jnp.tile(b1, 8).reshape(1, 128)
    b2w = jnp.tile(jnp.pad(b2, (0, F - b2.shape[0])), 8).reshape(1, 128)
    smat = jnp.asarray(_S_np)

    degw = _deg_kernel(ei)

    dinv, h1p = pl.pallas_call(
        _tc1_body,
        out_shape=(jax.ShapeDtypeStruct((WR, 128), jnp.float32),
                   jax.ShapeDtypeStruct((NP * F,), jnp.float32)),
    )(xw, w1blk, degw, smat)

    acc1 = _edge_kernel(ei, h1p.reshape(NP, F))

    h2p = pl.pallas_call(
        _tc2_body,
        out_shape=jax.ShapeDtypeStruct((NP * F,), jnp.float32),
    )(acc1.reshape(NC * NP * F), dinv, b1w, w2blk)

    acc2 = _edge_kernel(ei, h2p.reshape(NP, F))

    outw = pl.pallas_call(
        _tc3_body,
        out_shape=jax.ShapeDtypeStruct((NP * F,), jnp.float32),
    )(acc2.reshape(NC * NP * F), dinv, b2w)
    return outw.reshape(NP, F)[:N, :3]
